# baseline (speedup 1.0000x reference)
import functools
import math

import jax
import jax.numpy as jnp
from jax import lax
from jax.experimental import pallas as pl
from jax.experimental.pallas import tpu as pltpu
from jax.experimental.pallas import tpu_sc as plsc

N = 10000
E = 320000
C = 64
W = 128
G = 128
NCORES = 2
NSUB = 16
EPC = E // NCORES
EPT = EPC // NSUB
CH = 80
NCH = EPT // CH
NB = 5
NG = NCH // NB
ROWT = 1000
RCH = 200

F32 = jnp.float32


@functools.lru_cache(maxsize=None)
def _mesh():
    return plsc.VectorSubcoreMesh(core_axis_name="c", subcore_axis_name="s")


def _sc_gather_body(nf_hbm, kv_hbm, src_hbm, dst_hbm, gs_hbm, gd_hbm, *scr):
    idx_s = scr[0:NB]
    idx_d = scr[NB:2 * NB]
    buf_s = scr[2 * NB:3 * NB]
    buf_d = scr[3 * NB:4 * NB]
    sems = scr[4 * NB:]
    sem_is = sems[0:NB]
    sem_id = sems[NB:2 * NB]
    sem_gs = sems[2 * NB:3 * NB]
    sem_gd = sems[3 * NB:4 * NB]
    sem_ws = sems[4 * NB:5 * NB]
    sem_wd = sems[5 * NB:6 * NB]
    cid = lax.axis_index("c")
    sid = lax.axis_index("s")
    base = cid * EPC + sid * EPT

    def group(g, _):
        ic = []
        for b in range(NB):
            off = base + (g * NB + b) * CH
            ic.append(pltpu.async_copy(src_hbm.at[pl.ds(off, CH)],
                                       idx_s[b], sem_is[b]))
            ic.append(pltpu.async_copy(dst_hbm.at[pl.ds(off, CH)],
                                       idx_d[b], sem_id[b]))
        gc = []
        for b in range(NB):
            ic[2 * b].wait()
            ic[2 * b + 1].wait()
            gc.append(pltpu.async_copy(kv_hbm.at[idx_s[b]], buf_s[b],
                                       sem_gs[b]))
            gc.append(pltpu.async_copy(nf_hbm.at[idx_d[b]], buf_d[b],
                                       sem_gd[b]))
        wc = []
        for b in range(NB):
            off = base + (g * NB + b) * CH
            gc[2 * b].wait()
            gc[2 * b + 1].wait()
            wc.append(pltpu.async_copy(buf_s[b], gs_hbm.at[pl.ds(off, CH)],
                                       sem_ws[b]))
            wc.append(pltpu.async_copy(buf_d[b], gd_hbm.at[pl.ds(off, CH)],
                                       sem_wd[b]))
        for w in wc:
            w.wait()
        return 0

    lax.fori_loop(0, NG, group, 0)


@functools.lru_cache(maxsize=None)
def _sc_gather_kernel():
    return pl.kernel(
        _sc_gather_body,
        out_type=[jax.ShapeDtypeStruct((E, W), F32),
                  jax.ShapeDtypeStruct((E, W), F32)],
        mesh=_mesh(),
        scratch_types=(
            [pltpu.VMEM((CH,), jnp.int32) for _ in range(2 * NB)]
            + [pltpu.VMEM((CH, W), F32) for _ in range(2 * NB)]
            + [pltpu.SemaphoreType.DMA for _ in range(6 * NB)]
        ),
    )


def _sc_gather(nf, kvtab, src, dst):
    return _sc_gather_kernel()(nf, kvtab, src, dst)


def _sc_scatter_body(m_hbm, dst_hbm, zero_hbm, agg_hbm,
                     idx_v, mbuf, rbuf, acc_sh, sem):
    cid = lax.axis_index("c")
    sid = lax.axis_index("s")
    base = cid * EPC + sid * EPT

    @pl.when(sid < N // ROWT)
    def _zero():
        def zstep(j, _):
            r0 = sid * ROWT + j * RCH
            pltpu.sync_copy(zero_hbm.at[pl.ds(r0, RCH)], rbuf)
            pltpu.sync_copy(rbuf, acc_sh.at[pl.ds(r0, RCH)])
            return 0
        lax.fori_loop(0, ROWT // RCH, zstep, 0)

    plsc.subcore_barrier()

    def step(k, _):
        off = base + k * CH
        pltpu.sync_copy(dst_hbm.at[pl.ds(off, CH)], idx_v)
        pltpu.sync_copy(m_hbm.at[pl.ds(off, CH)], mbuf)
        pltpu.sync_copy(mbuf, acc_sh.at[idx_v], add=True)
        return 0

    lax.fori_loop(0, NCH, step, 0)
    plsc.subcore_barrier()

    @pl.when(sid < N // ROWT)
    def _out():
        def ostep(j, _):
            r0 = sid * ROWT + j * RCH
            pltpu.sync_copy(acc_sh.at[pl.ds(r0, RCH)], rbuf)
            pltpu.sync_copy(rbuf, agg_hbm.at[cid, pl.ds(r0, RCH)])
            return 0
        lax.fori_loop(0, ROWT // RCH, ostep, 0)


@functools.lru_cache(maxsize=None)
def _sc_scatter_kernel():
    return pl.kernel(
        _sc_scatter_body,
        out_type=jax.ShapeDtypeStruct((NCORES, N, W), F32),
        mesh=_mesh(),
        scratch_types=[
            pltpu.VMEM((CH,), jnp.int32),
            pltpu.VMEM((CH, W), F32),
            pltpu.VMEM((RCH, W), F32),
            pltpu.VMEM_SHARED((N, W), F32),
            pltpu.SemaphoreType.DMA,
        ],
    )


def _sc_scatter(m, dst, zeros_nc):
    return _sc_scatter_kernel()(m, dst, zeros_nc)


def _dot(a, b):
    return jnp.dot(a, b, preferred_element_type=F32)


def _ln(x, g, b):
    mu = jnp.mean(x, axis=-1, keepdims=True)
    var = jnp.mean((x - mu) ** 2, axis=-1, keepdims=True)
    return (x - mu) * lax.rsqrt(var + 1e-5) * g + b


def _pad_w(v):
    n = v.shape[0]
    return jnp.concatenate([v, jnp.zeros((n, W - C), F32)], axis=1)


def _atom_body(x_ref, w_ref, b_ref, o_ref):
    o_ref[...] = _pad_w(_dot(x_ref[...], w_ref[...]) + b_ref[...])


def _edgefeat_body(ea_ref, w0_ref, b0_ref, w2_ref, b2_ref, o_ref):
    h = _dot(ea_ref[...], w0_ref[...]) + b0_ref[...]
    beta = float(C)
    z = jnp.minimum(h * beta, beta)
    sp = jnp.where(h * beta > beta, h, jnp.log1p(jnp.exp(z)) / beta)
    o_ref[...] = _dot(sp, w2_ref[...]) + b2_ref[...]


def _prep_body(nf_ref, wkv_ref, bkv_ref, o_ref):
    o_ref[...] = _dot(nf_ref[:, :C], wkv_ref[...]) + bkv_ref[...]


EBLK = 2000


def _edge_body(gs_ref, gd_ref, ef_ref, wqkv_ref, bqkv_ref,
               we_ref, be_ref, wupd_ref, bupd_ref, wmsg_ref, bmsg_ref,
               lng_ref, lnb_ref, mlng_ref, mlnb_ref, m_ref):
    k_j = gs_ref[:, :C]
    v_j = gs_ref[:, C:]
    x_i = gd_ref[:, :C]
    qkv = _dot(x_i, wqkv_ref[...]) + bqkv_ref[...]
    e = _dot(ef_ref[...], we_ref[...]) + be_ref[...]
    q_i = qkv[:, :C]
    k_i = qkv[:, C:2 * C]
    v_i = qkv[:, 2 * C:]
    scale = 1.0 / math.sqrt(3.0 * C)
    alpha = jnp.concatenate([q_i * k_i, q_i * k_j, q_i * e], axis=1) * scale
    sig = jax.nn.sigmoid(_ln(alpha, lng_ref[...], lnb_ref[...]))
    vij = jnp.concatenate([v_i, v_j, e], axis=1)
    upd = _dot(vij, wupd_ref[...]) + bupd_ref[...]
    h = sig * upd
    msg = _dot(h, wmsg_ref[...]) + bmsg_ref[...]
    m_ref[...] = _pad_w(_ln(msg, mlng_ref[...], mlnb_ref[...]))


def _node_body(agg2_ref, nf_ref, wcat_ref, bcat_ref, bng_ref, bnb_ref,
               wfea_ref, bfea_ref, o_ref):
    agg = agg2_ref[0, :, :C] + agg2_ref[1, :, :C]
    o = _dot(agg, wcat_ref[...]) + bcat_ref[...]
    mu = jnp.mean(o, axis=0, keepdims=True)
    var = jnp.mean((o - mu) ** 2, axis=0, keepdims=True)
    o = (o - mu) * lax.rsqrt(var + 1e-5) * bng_ref[...] + bnb_ref[...]
    o = o * jax.nn.sigmoid(o)
    o_ref[...] = _pad_w(o + _dot(nf_ref[:, :C], wfea_ref[...]) + bfea_ref[...])


def _pool_body(nf_ref, b_ref, wfc_ref, bfc_ref, wout_ref, bout_ref, o_ref):
    nf = nf_ref[:, :C]
    onehot = (b_ref[...] == lax.broadcasted_iota(jnp.int32, (N, G), 1))
    onehot = onehot.astype(F32)
    seg = lax.dot_general(onehot, nf, (((0,), (0,)), ((), ())),
                          preferred_element_type=F32)
    cnt = jnp.sum(onehot, axis=0, keepdims=True).T
    pooled = seg / jnp.maximum(cnt, 1.0)
    h = _dot(pooled, wfc_ref[...]) + bfc_ref[...]
    h = h * jax.nn.sigmoid(h)
    o_ref[...] = _dot(h, wout_ref[...]) + bout_ref[...]


def _full(shape_like):
    return pl.BlockSpec(shape_like, lambda *_: tuple(0 for _ in shape_like))


def _tc_call(body, out_shape, in_specs, out_specs, grid):
    return pl.pallas_call(
        body,
        out_shape=out_shape,
        grid=grid,
        in_specs=in_specs,
        out_specs=out_specs,
    )


def kernel(x, edge_attr, edge_index, batch, params):
    src = edge_index[0].astype(jnp.int32)
    dst = edge_index[1].astype(jnp.int32)
    batch2 = batch.astype(jnp.int32).reshape(N, 1)
    zeros_nc = jnp.zeros((N, W), F32)

    p = params

    nf = _tc_call(
        _atom_body,
        jax.ShapeDtypeStruct((N, W), F32),
        [_full((N, x.shape[1])), _full((x.shape[1], C)), _full((1, C))],
        _full((N, W)),
        (1,),
    )(x, p['atom_w'].T, p['atom_b'].reshape(1, C))

    ef = _tc_call(
        _edgefeat_body,
        jax.ShapeDtypeStruct((E, C), F32),
        [pl.BlockSpec((EBLK, edge_attr.shape[1]), lambda i: (i, 0)),
         _full((edge_attr.shape[1], C)), _full((1, C)),
         _full((C, C)), _full((1, C))],
        pl.BlockSpec((EBLK, C), lambda i: (i, 0)),
        (E // EBLK,),
    )(edge_attr, p['e0_w'].T, p['e0_b'].reshape(1, C),
      p['e2_w'].T, p['e2_b'].reshape(1, C))

    for lp in p['layers']:
        wqkv = jnp.concatenate([lp['q_w'].T, lp['k_w'].T, lp['v_w'].T], axis=1)
        bqkv = jnp.concatenate([lp['q_b'], lp['k_b'], lp['v_b']]).reshape(1, 3 * C)
        wkv = jnp.concatenate([lp['k_w'].T, lp['v_w'].T], axis=1)
        bkv = jnp.concatenate([lp['k_b'], lp['v_b']]).reshape(1, 2 * C)

        kvtab = _tc_call(
            _prep_body,
            jax.ShapeDtypeStruct((N, W), F32),
            [_full((N, W)), _full((C, 2 * C)), _full((1, 2 * C))],
            _full((N, W)),
            (1,),
        )(nf, wkv, bkv)

        g_src, g_dst = _sc_gather(nf, kvtab, src, dst)

        m = _tc_call(
            _edge_body,
            jax.ShapeDtypeStruct((E, W), F32),
            [pl.BlockSpec((EBLK, W), lambda i: (i, 0)),
             pl.BlockSpec((EBLK, W), lambda i: (i, 0)),
             pl.BlockSpec((EBLK, C), lambda i: (i, 0)),
             _full((C, 3 * C)), _full((1, 3 * C)),
             _full((C, C)), _full((1, C)),
             _full((3 * C, 3 * C)), _full((1, 3 * C)),
             _full((3 * C, C)), _full((1, C)),
             _full((1, 3 * C)), _full((1, 3 * C)),
             _full((1, C)), _full((1, C))],
            pl.BlockSpec((EBLK, W), lambda i: (i, 0)),
            (E // EBLK,),
        )(g_src, g_dst, ef,
          wqkv, bqkv,
          lp['e_w'].T, lp['e_b'].reshape(1, C),
          lp['upd_w'].T, lp['upd_b'].reshape(1, 3 * C),
          lp['msg_w'].T, lp['msg_b'].reshape(1, C),
          lp['ln_g'].reshape(1, 3 * C), lp['ln_b'].reshape(1, 3 * C),
          lp['msg_ln_g'].reshape(1, C), lp['msg_ln_b'].reshape(1, C))

        agg2 = _sc_scatter(m, dst, zeros_nc)

        nf = _tc_call(
            _node_body,
            jax.ShapeDtypeStruct((N, W), F32),
            [_full((NCORES, N, W)), _full((N, W)),
             _full((C, C)), _full((1, C)), _full((1, C)), _full((1, C)),
             _full((C, C)), _full((1, C))],
            _full((N, W)),
            (1,),
        )(agg2, nf, lp['cat_w'].T, lp['cat_b'].reshape(1, C),
          lp['bn_g'].reshape(1, C), lp['bn_b'].reshape(1, C),
          lp['fea_w'].T, lp['fea_b'].reshape(1, C))

    out = _tc_call(
        _pool_body,
        jax.ShapeDtypeStruct((G, 1), F32),
        [_full((N, W)), _full((N, 1)),
         _full((C, 2 * C)), _full((1, 2 * C)),
         _full((2 * C, 1)), _full((1, 1))],
        _full((G, 1)),
        (1,),
    )(nf, batch2, p['fc_w'].T, p['fc_b'].reshape(1, 2 * C),
      p['out_w'].T, p['out_b'].reshape(1, 1))

    return out


# speedup vs baseline: 1.0397x; 1.0397x over previous
import functools
import math

import jax
import jax.numpy as jnp
from jax import lax
from jax.experimental import pallas as pl
from jax.experimental.pallas import tpu as pltpu
from jax.experimental.pallas import tpu_sc as plsc

N = 10000
E = 320000
C = 64
W = 128
G = 128
NCORES = 2
NSUB = 16
EPC = E // NCORES
EPT = EPC // NSUB
CH = 80
NCH = EPT // CH
NB = 5
NG = NCH // NB
ROWT = 1000
RCH = 200

F32 = jnp.float32


@functools.lru_cache(maxsize=None)
def _mesh():
    return plsc.VectorSubcoreMesh(core_axis_name="c", subcore_axis_name="s")


def _make_gather_body(ne, ch):
    epc = ne // NCORES
    ept = epc // NSUB
    ng = (ept // ch) // NB

    def body(nf_hbm, kv_hbm, src_hbm, dst_hbm, gs_hbm, gd_hbm, *scr):
        idx_s = scr[0:NB]
        idx_d = scr[NB:2 * NB]
        buf_s = scr[2 * NB:3 * NB]
        buf_d = scr[3 * NB:4 * NB]
        sems = scr[4 * NB:]
        sem_is = sems[0:NB]
        sem_id = sems[NB:2 * NB]
        sem_gs = sems[2 * NB:3 * NB]
        sem_gd = sems[3 * NB:4 * NB]
        sem_ws = sems[4 * NB:5 * NB]
        sem_wd = sems[5 * NB:6 * NB]
        cid = lax.axis_index("c")
        sid = lax.axis_index("s")
        base = cid * epc + sid * ept

        def group(g, _):
            ic = []
            for b in range(NB):
                off = base + (g * NB + b) * ch
                ic.append(pltpu.async_copy(src_hbm.at[pl.ds(off, ch)],
                                           idx_s[b], sem_is[b]))
                ic.append(pltpu.async_copy(dst_hbm.at[pl.ds(off, ch)],
                                           idx_d[b], sem_id[b]))
            gc = []
            for b in range(NB):
                ic[2 * b].wait()
                ic[2 * b + 1].wait()
                gc.append(pltpu.async_copy(kv_hbm.at[idx_s[b]], buf_s[b],
                                           sem_gs[b]))
                gc.append(pltpu.async_copy(nf_hbm.at[idx_d[b]], buf_d[b],
                                           sem_gd[b]))
            wc = []
            for b in range(NB):
                off = base + (g * NB + b) * ch
                gc[2 * b].wait()
                gc[2 * b + 1].wait()
                wc.append(pltpu.async_copy(buf_s[b],
                                           gs_hbm.at[pl.ds(off, ch)],
                                           sem_ws[b]))
                wc.append(pltpu.async_copy(buf_d[b],
                                           gd_hbm.at[pl.ds(off, ch)],
                                           sem_wd[b]))
            for w in wc:
                w.wait()
            return 0

        lax.fori_loop(0, ng, group, 0)

    return body


@functools.lru_cache(maxsize=None)
def _sc_gather_kernel(ne, ch):
    return pl.kernel(
        _make_gather_body(ne, ch),
        out_type=[jax.ShapeDtypeStruct((ne, W), F32),
                  jax.ShapeDtypeStruct((ne, W), F32)],
        mesh=_mesh(),
        scratch_types=(
            [pltpu.VMEM((ch,), jnp.int32) for _ in range(2 * NB)]
            + [pltpu.VMEM((ch, W), F32) for _ in range(2 * NB)]
            + [pltpu.SemaphoreType.DMA for _ in range(6 * NB)]
        ),
    )


def _sc_gather(nf, kvtab, src, dst, ch=CH):
    return _sc_gather_kernel(src.shape[0], ch)(nf, kvtab, src, dst)


def _make_scatter_body(ne, ch):
    epc = ne // NCORES
    ept = epc // NSUB
    nch = ept // ch

    def body(m_hbm, dst_hbm, zero_hbm, agg_hbm, idx_v, mbuf, rbuf, acc_sh,
             sem):
        cid = lax.axis_index("c")
        sid = lax.axis_index("s")
        base = cid * epc + sid * ept

        @pl.when(sid < N // ROWT)
        def _zero():
            def zstep(j, _):
                r0 = sid * ROWT + j * RCH
                pltpu.sync_copy(zero_hbm.at[pl.ds(r0, RCH)], rbuf)
                pltpu.sync_copy(rbuf, acc_sh.at[pl.ds(r0, RCH)])
                return 0
            lax.fori_loop(0, ROWT // RCH, zstep, 0)

        plsc.subcore_barrier()

        def step(k, _):
            off = base + k * ch
            pltpu.sync_copy(dst_hbm.at[pl.ds(off, ch)], idx_v)
            pltpu.sync_copy(m_hbm.at[pl.ds(off, ch)], mbuf)
            pltpu.sync_copy(mbuf, acc_sh.at[idx_v], add=True)
            return 0

        lax.fori_loop(0, nch, step, 0)
        plsc.subcore_barrier()

        @pl.when(sid < N // ROWT)
        def _out():
            def ostep(j, _):
                r0 = sid * ROWT + j * RCH
                pltpu.sync_copy(acc_sh.at[pl.ds(r0, RCH)], rbuf)
                pltpu.sync_copy(rbuf, agg_hbm.at[cid, pl.ds(r0, RCH)])
                return 0
            lax.fori_loop(0, ROWT // RCH, ostep, 0)

    return body


@functools.lru_cache(maxsize=None)
def _sc_scatter_kernel(ne, ch):
    return pl.kernel(
        _make_scatter_body(ne, ch),
        out_type=jax.ShapeDtypeStruct((NCORES, N, W), F32),
        mesh=_mesh(),
        scratch_types=[
            pltpu.VMEM((ch,), jnp.int32),
            pltpu.VMEM((ch, W), F32),
            pltpu.VMEM((RCH, W), F32),
            pltpu.VMEM_SHARED((N, W), F32),
            pltpu.SemaphoreType.DMA,
        ],
    )


def _sc_scatter(m, dst, zeros_nc, ch=CH):
    return _sc_scatter_kernel(dst.shape[0], ch)(m, dst, zeros_nc)


def _dot(a, b):
    return jnp.dot(a, b, preferred_element_type=F32)


def _ln(x, g, b):
    mu = jnp.mean(x, axis=-1, keepdims=True)
    var = jnp.mean((x - mu) ** 2, axis=-1, keepdims=True)
    return (x - mu) * lax.rsqrt(var + 1e-5) * g + b


def _pad_w(v):
    n = v.shape[0]
    return jnp.concatenate([v, jnp.zeros((n, W - C), F32)], axis=1)


def _atom_body(x_ref, w_ref, b_ref, o_ref):
    o_ref[...] = _pad_w(_dot(x_ref[...], w_ref[...]) + b_ref[...])


def _edgefeat_body(ea_ref, w0_ref, b0_ref, w2_ref, b2_ref, o_ref):
    h = _dot(ea_ref[...], w0_ref[...]) + b0_ref[...]
    beta = float(C)
    z = jnp.minimum(h * beta, beta)
    sp = jnp.where(h * beta > beta, h, jnp.log1p(jnp.exp(z)) / beta)
    o_ref[...] = _dot(sp, w2_ref[...]) + b2_ref[...]


def _prep_body(nf_ref, wkv_ref, bkv_ref, o_ref):
    o_ref[...] = _dot(nf_ref[:, :C], wkv_ref[...]) + bkv_ref[...]


EBLK = 2000


def _edge_body(gs_ref, gd_ref, ef_ref, wqkv_ref, bqkv_ref,
               we_ref, be_ref, wupd_ref, bupd_ref, wmsg_ref, bmsg_ref,
               lng_ref, lnb_ref, mlng_ref, mlnb_ref, m_ref):
    k_j = gs_ref[:, :C]
    v_j = gs_ref[:, C:]
    x_i = gd_ref[:, :C]
    qkv = _dot(x_i, wqkv_ref[...]) + bqkv_ref[...]
    e = _dot(ef_ref[...], we_ref[...]) + be_ref[...]
    q_i = qkv[:, :C]
    k_i = qkv[:, C:2 * C]
    v_i = qkv[:, 2 * C:]
    scale = 1.0 / math.sqrt(3.0 * C)
    alpha = jnp.concatenate([q_i * k_i, q_i * k_j, q_i * e], axis=1) * scale
    sig = jax.nn.sigmoid(_ln(alpha, lng_ref[...], lnb_ref[...]))
    vij = jnp.concatenate([v_i, v_j, e], axis=1)
    upd = _dot(vij, wupd_ref[...]) + bupd_ref[...]
    h = sig * upd
    msg = _dot(h, wmsg_ref[...]) + bmsg_ref[...]
    m_ref[...] = _pad_w(_ln(msg, mlng_ref[...], mlnb_ref[...]))


def _node_body(agg2a_ref, agg2b_ref, nf_ref, wcat_ref, bcat_ref,
               bng_ref, bnb_ref, wfea_ref, bfea_ref, o_ref):
    agg = (agg2a_ref[0, :, :C] + agg2a_ref[1, :, :C]
           + agg2b_ref[0, :, :C] + agg2b_ref[1, :, :C])
    o = _dot(agg, wcat_ref[...]) + bcat_ref[...]
    mu = jnp.mean(o, axis=0, keepdims=True)
    var = jnp.mean((o - mu) ** 2, axis=0, keepdims=True)
    o = (o - mu) * lax.rsqrt(var + 1e-5) * bng_ref[...] + bnb_ref[...]
    o = o * jax.nn.sigmoid(o)
    o_ref[...] = _pad_w(o + _dot(nf_ref[:, :C], wfea_ref[...]) + bfea_ref[...])


def _pool_body(nf_ref, b_ref, wfc_ref, bfc_ref, wout_ref, bout_ref, o_ref):
    nf = nf_ref[:, :C]
    onehot = (b_ref[...] == lax.broadcasted_iota(jnp.int32, (N, G), 1))
    onehot = onehot.astype(F32)
    seg = lax.dot_general(onehot, nf, (((0,), (0,)), ((), ())),
                          preferred_element_type=F32)
    cnt = jnp.sum(onehot, axis=0, keepdims=True).T
    pooled = seg / jnp.maximum(cnt, 1.0)
    h = _dot(pooled, wfc_ref[...]) + bfc_ref[...]
    h = h * jax.nn.sigmoid(h)
    o_ref[...] = _dot(h, wout_ref[...]) + bout_ref[...]


def _full(shape_like):
    return pl.BlockSpec(shape_like, lambda *_: tuple(0 for _ in shape_like))


def _tc_call(body, out_shape, in_specs, out_specs, grid):
    return pl.pallas_call(
        body,
        out_shape=out_shape,
        grid=grid,
        in_specs=in_specs,
        out_specs=out_specs,
    )


def kernel(x, edge_attr, edge_index, batch, params):
    src = edge_index[0].astype(jnp.int32)
    dst = edge_index[1].astype(jnp.int32)
    batch2 = batch.astype(jnp.int32).reshape(N, 1)
    zeros_nc = jnp.zeros((N, W), F32)

    p = params

    nf = _tc_call(
        _atom_body,
        jax.ShapeDtypeStruct((N, W), F32),
        [_full((N, x.shape[1])), _full((x.shape[1], C)), _full((1, C))],
        _full((N, W)),
        (1,),
    )(x, p['atom_w'].T, p['atom_b'].reshape(1, C))

    ef = _tc_call(
        _edgefeat_body,
        jax.ShapeDtypeStruct((E, C), F32),
        [pl.BlockSpec((EBLK, edge_attr.shape[1]), lambda i: (i, 0)),
         _full((edge_attr.shape[1], C)), _full((1, C)),
         _full((C, C)), _full((1, C))],
        pl.BlockSpec((EBLK, C), lambda i: (i, 0)),
        (E // EBLK,),
    )(edge_attr, p['e0_w'].T, p['e0_b'].reshape(1, C),
      p['e2_w'].T, p['e2_b'].reshape(1, C))

    for lp in p['layers']:
        wqkv = jnp.concatenate([lp['q_w'].T, lp['k_w'].T, lp['v_w'].T], axis=1)
        bqkv = jnp.concatenate([lp['q_b'], lp['k_b'], lp['v_b']]).reshape(1, 3 * C)
        wkv = jnp.concatenate([lp['k_w'].T, lp['v_w'].T], axis=1)
        bkv = jnp.concatenate([lp['k_b'], lp['v_b']]).reshape(1, 2 * C)

        kvtab = _tc_call(
            _prep_body,
            jax.ShapeDtypeStruct((N, W), F32),
            [_full((N, W)), _full((C, 2 * C)), _full((1, 2 * C))],
            _full((N, W)),
            (1,),
        )(nf, wkv, bkv)

        aggs = []
        for h in range(2):
            e0, e1 = h * (E // 2), (h + 1) * (E // 2)
            g_src, g_dst = _sc_gather(nf, kvtab, src[e0:e1], dst[e0:e1],
                                      ch=CH // 2)

            m = _tc_call(
                _edge_body,
                jax.ShapeDtypeStruct((E // 2, W), F32),
                [pl.BlockSpec((EBLK, W), lambda i: (i, 0)),
                 pl.BlockSpec((EBLK, W), lambda i: (i, 0)),
                 pl.BlockSpec((EBLK, C),
                              lambda i, h=h: (i + h * (E // 2 // EBLK), 0)),
                 _full((C, 3 * C)), _full((1, 3 * C)),
                 _full((C, C)), _full((1, C)),
                 _full((3 * C, 3 * C)), _full((1, 3 * C)),
                 _full((3 * C, C)), _full((1, C)),
                 _full((1, 3 * C)), _full((1, 3 * C)),
                 _full((1, C)), _full((1, C))],
                pl.BlockSpec((EBLK, W), lambda i: (i, 0)),
                (E // 2 // EBLK,),
            )(g_src, g_dst, ef,
              wqkv, bqkv,
              lp['e_w'].T, lp['e_b'].reshape(1, C),
              lp['upd_w'].T, lp['upd_b'].reshape(1, 3 * C),
              lp['msg_w'].T, lp['msg_b'].reshape(1, C),
              lp['ln_g'].reshape(1, 3 * C), lp['ln_b'].reshape(1, 3 * C),
              lp['msg_ln_g'].reshape(1, C), lp['msg_ln_b'].reshape(1, C))

            aggs.append(_sc_scatter(m, dst[e0:e1], zeros_nc, ch=CH // 2))

        nf = _tc_call(
            _node_body,
            jax.ShapeDtypeStruct((N, W), F32),
            [_full((NCORES, N, W)), _full((NCORES, N, W)), _full((N, W)),
             _full((C, C)), _full((1, C)), _full((1, C)), _full((1, C)),
             _full((C, C)), _full((1, C))],
            _full((N, W)),
            (1,),
        )(aggs[0], aggs[1], nf, lp['cat_w'].T, lp['cat_b'].reshape(1, C),
          lp['bn_g'].reshape(1, C), lp['bn_b'].reshape(1, C),
          lp['fea_w'].T, lp['fea_b'].reshape(1, C))

    out = _tc_call(
        _pool_body,
        jax.ShapeDtypeStruct((G, 1), F32),
        [_full((N, W)), _full((N, 1)),
         _full((C, 2 * C)), _full((1, 2 * C)),
         _full((2 * C, 1)), _full((1, 1))],
        _full((G, 1)),
        (1,),
    )(nf, batch2, p['fc_w'].T, p['fc_b'].reshape(1, 2 * C),
      p['out_w'].T, p['out_b'].reshape(1, 1))

    return out
